# baseline (speedup 1.0000x reference)
import functools

import numpy as np
import jax
import jax.numpy as jnp
from jax import lax
from jax.experimental import pallas as pl
from jax.experimental.pallas import tpu as pltpu
from jax.experimental.pallas import tpu_sc as plsc

_MAX_LEN = 2048
_D_MODEL = 1024

_NC, _NS, _L = 2, 16, 16
_NW = _NC * _NS
_PPC = 8
_NB = 3


def _pe_table(max_len, d_model):
    pos = np.arange(0, max_len, dtype=np.float64)[:, None]
    mul = np.exp(np.arange(0, d_model, 2, dtype=np.float64)
                 * -(np.log(10000.0) / d_model))
    pe = np.zeros((max_len, d_model), dtype=np.float64)
    pe[:, 0::2] = np.sin(pos * mul)
    pe[:, 1::2] = np.cos(pos * mul)
    return jnp.asarray(pe, dtype=jnp.float32)


_PE = _pe_table(_MAX_LEN, _D_MODEL)


def kernel(tokens, embed_table):
    B, S = tokens.shape
    V, D = embed_table.shape
    ppw = S // _NW
    n_chunks = ppw // _PPC
    groups = D // _L
    half = groups // 2

    tok = tokens.astype(jnp.int32)
    mesh = plsc.VectorSubcoreMesh(core_axis_name="c", subcore_axis_name="s")

    @functools.partial(
        pl.kernel,
        mesh=mesh,
        out_type=jax.ShapeDtypeStruct((B, S, D), jnp.float32),
        scratch_types=(
            [pltpu.VMEM((B, ppw), jnp.int32)]
            + [pltpu.VMEM((B, _PPC, D), jnp.float32)] * _NB
            + [pltpu.VMEM((_PPC, D), jnp.float32)] * _NB
            + [pltpu.SemaphoreType.DMA] * (3 * _NB)
        ),
    )
    def emb_kernel(table_h, tok_h, pe_h, out_h,
                   gidx_v, r0, r1, r2, p0, p1, p2,
                   g0, g1, g2, q0, q1, q2, s0, s1, s2):
        rows = [r0, r1, r2]
        pes = [p0, p1, p2]
        gsem = [g0, g1, g2]
        psem = [q0, q1, q2]
        ssem = [s0, s1, s2]
        wid = lax.axis_index("s") * _NC + lax.axis_index("c")
        pbase = wid * ppw
        for bb in range(B):
            pltpu.sync_copy(tok_h.at[bb, pl.ds(pbase, ppw)], gidx_v.at[bb])

        def start_chunk(c):
            b = c % _NB
            for bb in range(B):
                pltpu.async_copy(
                    table_h.at[gidx_v.at[bb, pl.ds(c * _PPC, _PPC)]],
                    rows[b].at[bb], gsem[b])
            pltpu.async_copy(pe_h.at[pl.ds(pbase + c * _PPC, _PPC)],
                             pes[b], psem[b])

        def wait_chunk(c):
            b = c % _NB
            for bb in range(B):
                pltpu.make_async_copy(
                    table_h.at[gidx_v.at[bb, pl.ds(c * _PPC, _PPC)]],
                    rows[b].at[bb], gsem[b]).wait()
            pltpu.make_async_copy(pe_h.at[pl.ds(pbase + c * _PPC, _PPC)],
                                  pes[b], psem[b]).wait()

        def start_scatter(c):
            b = c % _NB
            for bb in range(B):
                pltpu.async_copy(
                    rows[b].at[bb],
                    out_h.at[bb, pl.ds(pbase + c * _PPC, _PPC)], ssem[b])

        def wait_scatter(c):
            b = c % _NB
            for bb in range(B):
                pltpu.make_async_copy(
                    rows[b].at[bb],
                    out_h.at[bb, pl.ds(pbase + c * _PPC, _PPC)],
                    ssem[b]).wait()

        def add_chunk(c):
            b = c % _NB
            rv, pv = rows[b], pes[b]

            def body(t, _):
                i = t >> 1
                base = (t & 1) * (half * _L)
                for jg in range(half):
                    off = base + jg * _L
                    pe_reg = pv[i, pl.ds(off, _L)]
                    for bb in range(B):
                        rv[bb, i, pl.ds(off, _L)] = (
                            rv[bb, i, pl.ds(off, _L)] + pe_reg)
                return 0

            lax.fori_loop(0, _PPC * 2, body, 0)

        start_chunk(0)
        start_chunk(1)
        for c in range(n_chunks):
            wait_chunk(c)
            add_chunk(c)
            start_scatter(c)
            if c + 2 < n_chunks:
                if c >= 1:
                    wait_scatter(c - 1)
                start_chunk(c + 2)
        for c in range(n_chunks - _NB, n_chunks):
            wait_scatter(c)

    return emb_kernel(embed_table, tok, _PE)
